# baseline (speedup 1.0000x reference)
import functools

import jax
import jax.numpy as jnp
from jax import lax
from jax.experimental import pallas as pl
from jax.experimental.pallas import tpu as pltpu
from jax.experimental.pallas import tpu_sc as plsc

NCLASSES_ = 100000
EMB_D = 20
KB = EMB_D + 1
BATCH_ = 1024


_NC, _NS = 2, 16
_NW = _NC * _NS
_BPW = BATCH_ // _NW


def _sc_gather_t(table_t, idx):
    mesh = plsc.VectorSubcoreMesh(core_axis_name="c", subcore_axis_name="s")

    @functools.partial(
        pl.kernel,
        mesh=mesh,
        compiler_params=pltpu.CompilerParams(needs_layout_passes=False),
        out_type=jax.ShapeDtypeStruct((BATCH_, EMB_D), jnp.float32),
        scratch_types=[
            pltpu.VMEM((_BPW,), jnp.int32),
            pltpu.VMEM((_BPW * EMB_D, 128), jnp.float32),
            pltpu.VMEM((_BPW, EMB_D), jnp.float32),
            pltpu.SemaphoreType.DMA,
        ],
    )
    def gather_kernel(table_hbm, idx_hbm, out_hbm, idx_v, blocks_v, rows_v, sem):
        wid = lax.axis_index("s") * _NC + lax.axis_index("c")
        base = wid * _BPW
        pltpu.sync_copy(idx_hbm.at[pl.ds(base, _BPW)], idx_v)
        copies = []
        for g in range(_BPW // 16):
            vec = idx_v[pl.ds(g * 16, 16)]
            for i in range(16):
                c0 = pl.multiple_of((vec[i] >> 7) << 7, 128)
                copies.append(
                    pltpu.async_copy(
                        table_hbm.at[:, pl.ds(c0, 128)],
                        blocks_v.at[pl.ds((g * 16 + i) * EMB_D, EMB_D), :],
                        sem,
                    )
                )
        for c in copies:
            c.wait()
        lanes = lax.iota(jnp.int32, 16)
        for g in range(_BPW // 16):
            ivec = lanes + g * 16
            rvec = lax.rem(idx_v[pl.ds(g * 16, 16)], 128)
            for k in range(EMB_D):
                ksplat = jnp.full((16,), k, jnp.int32)
                val = plsc.load_gather(blocks_v, [ivec * EMB_D + k, rvec])
                plsc.store_scatter(rows_v, [ivec, ksplat], val)
        pltpu.sync_copy(rows_v, out_hbm.at[pl.ds(base, _BPW)])

    return gather_kernel(table_t, idx)


_VT = 6144


def _mm_kernel(wb_ref, x_ref, o_ref):
    o_ref[...] = lax.dot_general(
        wb_ref[...], x_ref[...],
        (((1,), (1,)), ((), ())),
        preferred_element_type=jnp.float32,
    )


def _tc_linear_t(wb, xb):
    grid = pl.cdiv(NCLASSES_, _VT)
    return pl.pallas_call(
        _mm_kernel,
        grid=(grid,),
        in_specs=[
            pl.BlockSpec((_VT, KB), lambda j: (j, 0)),
            pl.BlockSpec((BATCH_, KB), lambda j: (0, 0)),
        ],
        out_specs=pl.BlockSpec((_VT, BATCH_), lambda j: (j, 0)),
        out_shape=jax.ShapeDtypeStruct((NCLASSES_, BATCH_), jnp.float32),
    )(wb, xb)


def kernel(input, emb_table, W, b):
    idx = input.astype(jnp.int32)
    x = _sc_gather_t(emb_table.T, idx)
    xb = jnp.concatenate(
        [x, jnp.ones((BATCH_, 1), jnp.float32)], axis=1
    ).astype(jnp.bfloat16)
    wb = jnp.concatenate([W, b[:, None]], axis=1).astype(jnp.bfloat16)
    out_t = _tc_linear_t(wb, xb)
    return out_t.T


# speedup vs baseline: 1.0012x; 1.0012x over previous
import functools

import jax
import jax.numpy as jnp
from jax import lax
from jax.experimental import pallas as pl
from jax.experimental.pallas import tpu as pltpu
from jax.experimental.pallas import tpu_sc as plsc

NCLASSES_ = 100000
EMB_D = 20
KB = EMB_D + 1
BATCH_ = 1024


_NC, _NS = 2, 16
_NW = _NC * _NS
_BPW = BATCH_ // _NW


def _sc_gather_t(table_t, idx):
    mesh = plsc.VectorSubcoreMesh(core_axis_name="c", subcore_axis_name="s")

    @functools.partial(
        pl.kernel,
        mesh=mesh,
        compiler_params=pltpu.CompilerParams(needs_layout_passes=False),
        out_type=jax.ShapeDtypeStruct((BATCH_, EMB_D), jnp.float32),
        scratch_types=[
            pltpu.VMEM((_BPW,), jnp.int32),
            pltpu.VMEM((_BPW * EMB_D, 128), jnp.float32),
            pltpu.VMEM((_BPW, EMB_D), jnp.float32),
            pltpu.SemaphoreType.DMA,
        ],
    )
    def gather_kernel(table_hbm, idx_hbm, out_hbm, idx_v, blocks_v, rows_v, sem):
        wid = lax.axis_index("s") * _NC + lax.axis_index("c")
        base = wid * _BPW
        pltpu.sync_copy(idx_hbm.at[pl.ds(base, _BPW)], idx_v)
        copies = []
        for g in range(_BPW // 16):
            vec = idx_v[pl.ds(g * 16, 16)]
            for i in range(16):
                c0 = pl.multiple_of((vec[i] >> 7) << 7, 128)
                copies.append(
                    pltpu.async_copy(
                        table_hbm.at[:, pl.ds(c0, 128)],
                        blocks_v.at[pl.ds((g * 16 + i) * EMB_D, EMB_D), :],
                        sem,
                    )
                )
        for c in copies:
            c.wait()
        lanes = lax.iota(jnp.int32, 16)
        for g in range(_BPW // 16):
            ivec = lanes + g * 16
            rvec = lax.rem(idx_v[pl.ds(g * 16, 16)], 128)
            for k in range(EMB_D):
                ksplat = jnp.full((16,), k, jnp.int32)
                val = plsc.load_gather(blocks_v, [ivec * EMB_D + k, rvec])
                plsc.store_scatter(rows_v, [ivec, ksplat], val)
        pltpu.sync_copy(rows_v, out_hbm.at[pl.ds(base, _BPW)])

    return gather_kernel(table_t, idx)


_VT = 4096


def _mm_kernel(wb_ref, x_ref, o_ref):
    o_ref[...] = lax.dot_general(
        wb_ref[...], x_ref[...],
        (((1,), (1,)), ((), ())),
        preferred_element_type=jnp.float32,
    )


def _tc_linear_t(wb, xb):
    grid = pl.cdiv(NCLASSES_, _VT)
    return pl.pallas_call(
        _mm_kernel,
        grid=(grid,),
        in_specs=[
            pl.BlockSpec((_VT, KB), lambda j: (j, 0)),
            pl.BlockSpec((BATCH_, KB), lambda j: (0, 0)),
        ],
        out_specs=pl.BlockSpec((_VT, BATCH_), lambda j: (j, 0)),
        out_shape=jax.ShapeDtypeStruct((NCLASSES_, BATCH_), jnp.float32),
    )(wb, xb)


def kernel(input, emb_table, W, b):
    idx = input.astype(jnp.int32)
    x = _sc_gather_t(emb_table.T, idx)
    xb = jnp.concatenate(
        [x, jnp.ones((BATCH_, 1), jnp.float32)], axis=1
    ).astype(jnp.bfloat16)
    wb = jnp.concatenate([W, b[:, None]], axis=1).astype(jnp.bfloat16)
    out_t = _tc_linear_t(wb, xb)
    return out_t.T
